# baseline (speedup 1.0000x reference)
import functools

import jax
import jax.numpy as jnp
from jax import lax
from jax.experimental import pallas as pl
from jax.experimental.pallas import tpu as pltpu
from jax.experimental.pallas import tpu_sc as plsc

_B = 1024
_L = 200
_FACT = 64
_EMB = 128
_TOKENS = _B * _L
_NC = 2
_NS = 16
_NW = _NC * _NS
_IDX_W = 128
_IDXROWS = _TOKENS // _IDX_W
_IDXROWS_PER_W = _IDXROWS // _NW
_ROWS_PER_W = _TOKENS // _NW
_K = 5
_GROUPS = _IDXROWS_PER_W // _K
_LANES = 16


@functools.partial(
    pl.kernel,
    out_type=jax.ShapeDtypeStruct((_TOKENS, 128), jnp.float32),
    mesh=plsc.VectorSubcoreMesh(
        core_axis_name="c", subcore_axis_name="s",
        num_cores=_NC, num_subcores=_NS),
    scratch_types=[
        pltpu.VMEM((_IDXROWS_PER_W, _IDX_W), jnp.int32),
        pltpu.VMEM((_K * _IDX_W, _EMB), jnp.float32),
        pltpu.SemaphoreType.DMA,
        pltpu.SemaphoreType.DMA,
    ],
)
def _sc_gather(table_hbm, tok_hbm, out_hbm, idx_v, sup_v, gsem, ssem):
    wid = lax.axis_index("c") * _NS + lax.axis_index("s")
    pltpu.sync_copy(tok_hbm.at[wid], idx_v)

    def to_superrow(r, carry):
        for k in range(_IDX_W // _LANES):
            sl = pl.ds(k * _LANES, _LANES)
            v = idx_v[r, sl]
            idx_v[r, sl] = lax.shift_left(
                lax.shift_right_logical(v, 10), 9) | (v & 511)
        return carry

    lax.fori_loop(0, _IDXROWS_PER_W, to_superrow, 0)

    def group(grp, carry):
        j0 = grp * _K
        gh = []
        for b in range(_K):
            gh.append(pltpu.async_copy(
                table_hbm.at[idx_v.at[j0 + b]],
                sup_v.at[pl.ds(b * _IDX_W, _IDX_W)],
                gsem))
        for h in gh:
            h.wait()
        sh = []
        for b in range(_K):
            sh.append(pltpu.async_copy(
                sup_v.at[pl.ds(b * _IDX_W, _IDX_W)],
                out_hbm.at[pl.ds(wid * _ROWS_PER_W + (j0 + b) * _IDX_W,
                                 _IDX_W)],
                ssem))
        for h in sh:
            h.wait()
        return carry

    lax.fori_loop(0, _GROUPS, group, 0)


_VOCAB = 1000000
_VCHUNK = 1024
_TGRID = -(-_VOCAB // _VCHUNK)
_TROWS = _TGRID * _VCHUNK // 2


def _tx_body(a_ref, b_ref, o_ref):
    z = jnp.concatenate([a_ref[...], b_ref[...]], axis=0)
    o_ref[...] = z.T


_tc_transpose = pl.pallas_call(
    _tx_body,
    grid=(_TGRID,),
    in_specs=[
        pl.BlockSpec((_FACT, _VCHUNK // 2), lambda i: (0, 2 * i)),
        pl.BlockSpec((_FACT, _VCHUNK // 2), lambda i: (0, 2 * i + 1)),
    ],
    out_specs=pl.BlockSpec((_VCHUNK // 2, 2 * _FACT), lambda i: (i, 0)),
    out_shape=jax.ShapeDtypeStruct((_TROWS, 2 * _FACT), jnp.float32),
)


_BS = 16


def _tc_body(sup_ref, tok_ref, typ_ref, w_ref, b_ref, seg_ref, pos_ref,
             o_ref):
    sup = sup_ref[...]
    par = (lax.shift_right_logical(tok_ref[...], 9) & 1)[:, :, None]
    g = jnp.where(par == 1, sup[:, :, _FACT:], sup[:, :, :_FACT])
    mm = jnp.dot(g.reshape(_BS * _L, _FACT), w_ref[...],
                 preferred_element_type=jnp.float32)
    mm = mm.reshape(_BS, _L, _EMB)
    t = typ_ref[...].astype(jnp.float32)[:, :, None]
    s0 = seg_ref[0:1, :].reshape(1, 1, _EMB)
    s1 = seg_ref[1:2, :].reshape(1, 1, _EMB)
    pos = pos_ref[...][None, :, :]
    bias = b_ref[...].reshape(1, 1, _EMB)
    o_ref[...] = mm + bias + pos + s0 + t * (s1 - s0)


_tc_project = pl.pallas_call(
    _tc_body,
    grid=(_B // _BS,),
    in_specs=[
        pl.BlockSpec((_BS, _L, _EMB), lambda i: (i, 0, 0)),
        pl.BlockSpec((_BS, _L), lambda i: (i, 0)),
        pl.BlockSpec((_BS, _L), lambda i: (i, 0)),
        pl.BlockSpec((_FACT, _EMB), lambda i: (0, 0)),
        pl.BlockSpec((1, _EMB), lambda i: (0, 0)),
        pl.BlockSpec((2, _EMB), lambda i: (0, 0)),
        pl.BlockSpec((_L, _EMB), lambda i: (0, 0)),
    ],
    out_specs=pl.BlockSpec((_BS, _L, _EMB), lambda i: (i, 0, 0)),
    out_shape=jax.ShapeDtypeStruct((_B, _L, _EMB), jnp.float32),
)


def kernel(token_ids, type_token_ids, attention_mask, tok_table, W, b,
           seg_table, pos_table):
    table_t = tok_table.T
    table2 = _tc_transpose(table_t, table_t)
    tokens3d = token_ids.reshape(_NW, _IDXROWS_PER_W, _IDX_W)
    packed = _sc_gather(table2, tokens3d)
    p3 = packed.reshape(_B, _L, _EMB)
    out = _tc_project(p3, token_ids, type_token_ids, W, b.reshape(1, _EMB),
                      seg_table, pos_table)
    return (out, attention_mask)


# speedup vs baseline: 1.7361x; 1.7361x over previous
import functools

import jax
import jax.numpy as jnp
from jax import lax
from jax.experimental import pallas as pl
from jax.experimental.pallas import tpu as pltpu
from jax.experimental.pallas import tpu_sc as plsc

_B = 1024
_L = 200
_FACT = 64
_EMB = 128
_TOKENS = _B * _L
_NC = 2
_NS = 16
_NW = _NC * _NS
_IDX_W = 128
_IDXROWS = _TOKENS // _IDX_W
_IDXROWS_PER_W = _IDXROWS // _NW
_ROWS_PER_W = _TOKENS // _NW
_K = 5
_GROUPS = _IDXROWS_PER_W // _K
_LANES = 16


@functools.partial(
    pl.kernel,
    out_type=jax.ShapeDtypeStruct((_TOKENS, 128), jnp.float32),
    mesh=plsc.VectorSubcoreMesh(
        core_axis_name="c", subcore_axis_name="s",
        num_cores=_NC, num_subcores=_NS),
    scratch_types=[
        pltpu.VMEM((_IDXROWS_PER_W, _IDX_W), jnp.int32),
        pltpu.VMEM((_K * _IDX_W, _EMB), jnp.float32),
        pltpu.SemaphoreType.DMA,
        pltpu.SemaphoreType.DMA,
    ],
)
def _sc_gather(table_hbm, tok_hbm, out_hbm, idx_v, sup_v, gsem, ssem):
    wid = lax.axis_index("c") * _NS + lax.axis_index("s")
    pltpu.sync_copy(tok_hbm.at[wid], idx_v)

    def to_superrow(r, carry):
        for k in range(_IDX_W // _LANES):
            sl = pl.ds(k * _LANES, _LANES)
            v = idx_v[r, sl]
            idx_v[r, sl] = lax.shift_left(
                lax.shift_right_logical(v, _VSH), _HSH) | (v & _HMASK)
        return carry

    lax.fori_loop(0, _IDXROWS_PER_W, to_superrow, 0)

    def group(grp, carry):
        j0 = grp * _K
        gh = []
        for b in range(_K):
            gh.append(pltpu.async_copy(
                table_hbm.at[idx_v.at[j0 + b]],
                sup_v.at[pl.ds(b * _IDX_W, _IDX_W)],
                gsem))
        for h in gh:
            h.wait()
        sh = []
        for b in range(_K):
            sh.append(pltpu.async_copy(
                sup_v.at[pl.ds(b * _IDX_W, _IDX_W)],
                out_hbm.at[pl.ds(wid * _ROWS_PER_W + (j0 + b) * _IDX_W,
                                 _IDX_W)],
                ssem))
        for h in sh:
            h.wait()
        return carry

    lax.fori_loop(0, _GROUPS, group, 0)


_VOCAB = 1000000
_VCHUNK = 4096
_TGRID = -(-_VOCAB // _VCHUNK)
_TROWS = _TGRID * _VCHUNK // 2
_VSH = _VCHUNK.bit_length() - 1
_HSH = _VSH - 1
_HMASK = (1 << _HSH) - 1


def _tx_body(a_ref, b_ref, o_ref):
    z = jnp.concatenate([a_ref[...], b_ref[...]], axis=0)
    o_ref[...] = z.T


_tc_transpose = pl.pallas_call(
    _tx_body,
    grid=(_TGRID,),
    in_specs=[
        pl.BlockSpec((_FACT, _VCHUNK // 2), lambda i: (0, 2 * i)),
        pl.BlockSpec((_FACT, _VCHUNK // 2),
                     lambda i: (0, jnp.minimum(
                         2 * i + 1, _VOCAB // (_VCHUNK // 2) - 1))),
    ],
    out_specs=pl.BlockSpec((_VCHUNK // 2, 2 * _FACT), lambda i: (i, 0)),
    out_shape=jax.ShapeDtypeStruct((_TROWS, 2 * _FACT), jnp.float32),
)


_BS = 16


def _tc_body(sup_ref, tok_ref, typ_ref, w_ref, b_ref, seg_ref, pos_ref,
             o_ref):
    sup = sup_ref[...]
    par = (lax.shift_right_logical(tok_ref[...], _HSH) & 1)[:, :, None]
    g = jnp.where(par == 1, sup[:, :, _FACT:], sup[:, :, :_FACT])
    mm = jnp.dot(g.reshape(_BS * _L, _FACT), w_ref[...],
                 preferred_element_type=jnp.float32)
    mm = mm.reshape(_BS, _L, _EMB)
    t = typ_ref[...].astype(jnp.float32)[:, :, None]
    s0 = seg_ref[0:1, :].reshape(1, 1, _EMB)
    s1 = seg_ref[1:2, :].reshape(1, 1, _EMB)
    pos = pos_ref[...][None, :, :]
    bias = b_ref[...].reshape(1, 1, _EMB)
    o_ref[...] = mm + bias + pos + s0 + t * (s1 - s0)


_tc_project = pl.pallas_call(
    _tc_body,
    grid=(_B // _BS,),
    in_specs=[
        pl.BlockSpec((_BS, _L, _EMB), lambda i: (i, 0, 0)),
        pl.BlockSpec((_BS, _L), lambda i: (i, 0)),
        pl.BlockSpec((_BS, _L), lambda i: (i, 0)),
        pl.BlockSpec((_FACT, _EMB), lambda i: (0, 0)),
        pl.BlockSpec((1, _EMB), lambda i: (0, 0)),
        pl.BlockSpec((2, _EMB), lambda i: (0, 0)),
        pl.BlockSpec((_L, _EMB), lambda i: (0, 0)),
    ],
    out_specs=pl.BlockSpec((_BS, _L, _EMB), lambda i: (i, 0, 0)),
    out_shape=jax.ShapeDtypeStruct((_B, _L, _EMB), jnp.float32),
)


def kernel(token_ids, type_token_ids, attention_mask, tok_table, W, b,
           seg_table, pos_table):
    table_t = tok_table.T
    table2 = _tc_transpose(table_t, table_t)
    tokens3d = token_ids.reshape(_NW, _IDXROWS_PER_W, _IDX_W)
    packed = _sc_gather(table2, tokens3d)
    p3 = packed.reshape(_B, _L, _EMB)
    out = _tc_project(p3, token_ids, type_token_ids, W, b.reshape(1, _EMB),
                      seg_table, pos_table)
    return (out, attention_mask)


# speedup vs baseline: 2.1834x; 1.2577x over previous
import functools

import jax
import jax.numpy as jnp
from jax import lax
from jax.experimental import pallas as pl
from jax.experimental.pallas import tpu as pltpu
from jax.experimental.pallas import tpu_sc as plsc

_B = 1024
_L = 200
_FACT = 64
_EMB = 128
_TOKENS = _B * _L
_NC = 2
_NS = 16
_NW = _NC * _NS
_IDX_W = 128
_IDXROWS = _TOKENS // _IDX_W
_IDXROWS_PER_W = _IDXROWS // _NW
_ROWS_PER_W = _TOKENS // _NW
_K = 5
_GROUPS = _IDXROWS_PER_W // _K
_LANES = 16


@functools.partial(
    pl.kernel,
    out_type=jax.ShapeDtypeStruct((_TOKENS, 128), jnp.float32),
    mesh=plsc.VectorSubcoreMesh(
        core_axis_name="c", subcore_axis_name="s",
        num_cores=_NC, num_subcores=_NS),
    scratch_types=[
        pltpu.VMEM((_IDXROWS_PER_W, _IDX_W), jnp.int32),
        pltpu.VMEM((_K * _IDX_W, _EMB), jnp.float32),
        pltpu.SemaphoreType.DMA,
        pltpu.SemaphoreType.DMA,
    ],
)
def _sc_gather(table_hbm, tok_hbm, out_hbm, idx_v, sup_v, gsem, ssem):
    wid = lax.axis_index("c") * _NS + lax.axis_index("s")
    pltpu.sync_copy(tok_hbm.at[wid], idx_v)

    def to_superrow(r, carry):
        for k in range(_IDX_W // _LANES):
            sl = pl.ds(k * _LANES, _LANES)
            v = idx_v[r, sl]
            idx_v[r, sl] = lax.shift_left(
                lax.shift_right_logical(v, _VSH), _HSH) | (v & _HMASK)
        return carry

    lax.fori_loop(0, _IDXROWS_PER_W, to_superrow, 0)

    def group(grp, carry):
        j0 = grp * _K
        gh = []
        for b in range(_K):
            gh.append(pltpu.async_copy(
                table_hbm.at[idx_v.at[j0 + b]],
                sup_v.at[pl.ds(b * _IDX_W, _IDX_W)],
                gsem))
        for h in gh:
            h.wait()
        sh = []
        for b in range(_K):
            sh.append(pltpu.async_copy(
                sup_v.at[pl.ds(b * _IDX_W, _IDX_W)],
                out_hbm.at[pl.ds(wid * _ROWS_PER_W + (j0 + b) * _IDX_W,
                                 _IDX_W)],
                ssem))
        for h in sh:
            h.wait()
        return carry

    lax.fori_loop(0, _GROUPS, group, 0)


_VOCAB = 1000000
_VCHUNK = 16384
_TGRID = -(-_VOCAB // _VCHUNK)
_TROWS = _TGRID * _VCHUNK // 2
_VSH = _VCHUNK.bit_length() - 1
_HSH = _VSH - 1
_HMASK = (1 << _HSH) - 1


def _tx_body(a_ref, b_ref, o_ref):
    z = jnp.concatenate([a_ref[...], b_ref[...]], axis=0)
    o_ref[...] = z.T


_tc_transpose = pl.pallas_call(
    _tx_body,
    grid=(_TGRID,),
    in_specs=[
        pl.BlockSpec((_FACT, _VCHUNK // 2), lambda i: (0, 2 * i)),
        pl.BlockSpec((_FACT, _VCHUNK // 2),
                     lambda i: (0, jnp.minimum(
                         2 * i + 1, _VOCAB // (_VCHUNK // 2) - 1))),
    ],
    out_specs=pl.BlockSpec((_VCHUNK // 2, 2 * _FACT), lambda i: (i, 0)),
    out_shape=jax.ShapeDtypeStruct((_TROWS, 2 * _FACT), jnp.float32),
)


_BS = 16


def _tc_body(sup_ref, tok_ref, typ_ref, w_ref, b_ref, seg_ref, pos_ref,
             o_ref):
    sup = sup_ref[...]
    par = (lax.shift_right_logical(tok_ref[...], _HSH) & 1)[:, :, None]
    g = jnp.where(par == 1, sup[:, :, _FACT:], sup[:, :, :_FACT])
    mm = jnp.dot(g.reshape(_BS * _L, _FACT), w_ref[...],
                 preferred_element_type=jnp.float32)
    mm = mm.reshape(_BS, _L, _EMB)
    t = typ_ref[...].astype(jnp.float32)[:, :, None]
    s0 = seg_ref[0:1, :].reshape(1, 1, _EMB)
    s1 = seg_ref[1:2, :].reshape(1, 1, _EMB)
    pos = pos_ref[...][None, :, :]
    bias = b_ref[...].reshape(1, 1, _EMB)
    o_ref[...] = mm + bias + pos + s0 + t * (s1 - s0)


_tc_project = pl.pallas_call(
    _tc_body,
    grid=(_B // _BS,),
    in_specs=[
        pl.BlockSpec((_BS, _L, _EMB), lambda i: (i, 0, 0)),
        pl.BlockSpec((_BS, _L), lambda i: (i, 0)),
        pl.BlockSpec((_BS, _L), lambda i: (i, 0)),
        pl.BlockSpec((_FACT, _EMB), lambda i: (0, 0)),
        pl.BlockSpec((1, _EMB), lambda i: (0, 0)),
        pl.BlockSpec((2, _EMB), lambda i: (0, 0)),
        pl.BlockSpec((_L, _EMB), lambda i: (0, 0)),
    ],
    out_specs=pl.BlockSpec((_BS, _L, _EMB), lambda i: (i, 0, 0)),
    out_shape=jax.ShapeDtypeStruct((_B, _L, _EMB), jnp.float32),
)


def kernel(token_ids, type_token_ids, attention_mask, tok_table, W, b,
           seg_table, pos_table):
    table_t = tok_table.T
    table2 = _tc_transpose(table_t, table_t)
    tokens3d = token_ids.reshape(_NW, _IDXROWS_PER_W, _IDX_W)
    packed = _sc_gather(table2, tokens3d)
    p3 = packed.reshape(_B, _L, _EMB)
    out = _tc_project(p3, token_ids, type_token_ids, W, b.reshape(1, _EMB),
                      seg_table, pos_table)
    return (out, attention_mask)


# speedup vs baseline: 2.1979x; 1.0067x over previous
import functools

import jax
import jax.numpy as jnp
from jax import lax
from jax.experimental import pallas as pl
from jax.experimental.pallas import tpu as pltpu
from jax.experimental.pallas import tpu_sc as plsc

_B = 1024
_L = 200
_FACT = 64
_EMB = 128
_TOKENS = _B * _L
_NC = 2
_NS = 16
_NW = _NC * _NS
_IDX_W = 128
_IDXROWS = _TOKENS // _IDX_W
_IDXROWS_PER_W = _IDXROWS // _NW
_ROWS_PER_W = _TOKENS // _NW
_K = 5
_GROUPS = _IDXROWS_PER_W // _K
_LANES = 16


@functools.partial(
    pl.kernel,
    out_type=jax.ShapeDtypeStruct((_TOKENS, 128), jnp.float32),
    mesh=plsc.VectorSubcoreMesh(
        core_axis_name="c", subcore_axis_name="s",
        num_cores=_NC, num_subcores=_NS),
    scratch_types=[
        pltpu.VMEM((_IDXROWS_PER_W, _IDX_W), jnp.int32),
        pltpu.VMEM((_K * _IDX_W, _EMB), jnp.float32),
        pltpu.SemaphoreType.DMA,
        pltpu.SemaphoreType.DMA,
    ],
)
def _sc_gather(table_hbm, tok_hbm, out_hbm, idx_v, sup_v, gsem, ssem):
    wid = lax.axis_index("c") * _NS + lax.axis_index("s")
    pltpu.sync_copy(tok_hbm.at[wid], idx_v)

    def to_superrow(r, carry):
        for k in range(_IDX_W // _LANES):
            sl = pl.ds(k * _LANES, _LANES)
            v = idx_v[r, sl]
            idx_v[r, sl] = lax.shift_left(
                lax.shift_right_logical(v, _VSH), _HSH) | (v & _HMASK)
        return carry

    lax.fori_loop(0, _IDXROWS_PER_W, to_superrow, 0)

    def group(grp, carry):
        j0 = grp * _K
        gh = []
        for b in range(_K):
            gh.append(pltpu.async_copy(
                table_hbm.at[idx_v.at[j0 + b]],
                sup_v.at[pl.ds(b * _IDX_W, _IDX_W)],
                gsem))
        for h in gh:
            h.wait()
        sh = []
        for b in range(_K):
            sh.append(pltpu.async_copy(
                sup_v.at[pl.ds(b * _IDX_W, _IDX_W)],
                out_hbm.at[pl.ds(wid * _ROWS_PER_W + (j0 + b) * _IDX_W,
                                 _IDX_W)],
                ssem))
        for h in sh:
            h.wait()
        return carry

    lax.fori_loop(0, _GROUPS, group, 0)


_VOCAB = 1000000
_VCHUNK = 32768
_TGRID = -(-_VOCAB // _VCHUNK)
_TROWS = _TGRID * _VCHUNK // 2
_VSH = _VCHUNK.bit_length() - 1
_HSH = _VSH - 1
_HMASK = (1 << _HSH) - 1


def _tx_body(a_ref, b_ref, o_ref):
    z = jnp.concatenate([a_ref[...], b_ref[...]], axis=0)
    o_ref[...] = z.T


_tc_transpose = pl.pallas_call(
    _tx_body,
    grid=(_TGRID,),
    in_specs=[
        pl.BlockSpec((_FACT, _VCHUNK // 2), lambda i: (0, 2 * i)),
        pl.BlockSpec((_FACT, _VCHUNK // 2),
                     lambda i: (0, jnp.minimum(
                         2 * i + 1, _VOCAB // (_VCHUNK // 2) - 1))),
    ],
    out_specs=pl.BlockSpec((_VCHUNK // 2, 2 * _FACT), lambda i: (i, 0)),
    out_shape=jax.ShapeDtypeStruct((_TROWS, 2 * _FACT), jnp.float32),
)


_BS = 16


def _tc_body(sup_ref, tok_ref, typ_ref, w_ref, b_ref, seg_ref, pos_ref,
             o_ref):
    sup = sup_ref[...]
    par = (lax.shift_right_logical(tok_ref[...], _HSH) & 1)[:, :, None]
    g = jnp.where(par == 1, sup[:, :, _FACT:], sup[:, :, :_FACT])
    mm = jnp.dot(g.reshape(_BS * _L, _FACT), w_ref[...],
                 preferred_element_type=jnp.float32)
    mm = mm.reshape(_BS, _L, _EMB)
    t = typ_ref[...].astype(jnp.float32)[:, :, None]
    s0 = seg_ref[0:1, :].reshape(1, 1, _EMB)
    s1 = seg_ref[1:2, :].reshape(1, 1, _EMB)
    pos = pos_ref[...][None, :, :]
    bias = b_ref[...].reshape(1, 1, _EMB)
    o_ref[...] = mm + bias + pos + s0 + t * (s1 - s0)


_tc_project = pl.pallas_call(
    _tc_body,
    grid=(_B // _BS,),
    in_specs=[
        pl.BlockSpec((_BS, _L, _EMB), lambda i: (i, 0, 0)),
        pl.BlockSpec((_BS, _L), lambda i: (i, 0)),
        pl.BlockSpec((_BS, _L), lambda i: (i, 0)),
        pl.BlockSpec((_FACT, _EMB), lambda i: (0, 0)),
        pl.BlockSpec((1, _EMB), lambda i: (0, 0)),
        pl.BlockSpec((2, _EMB), lambda i: (0, 0)),
        pl.BlockSpec((_L, _EMB), lambda i: (0, 0)),
    ],
    out_specs=pl.BlockSpec((_BS, _L, _EMB), lambda i: (i, 0, 0)),
    out_shape=jax.ShapeDtypeStruct((_B, _L, _EMB), jnp.float32),
)


def kernel(token_ids, type_token_ids, attention_mask, tok_table, W, b,
           seg_table, pos_table):
    table_t = tok_table.T
    table2 = _tc_transpose(table_t, table_t)
    tokens3d = token_ids.reshape(_NW, _IDXROWS_PER_W, _IDX_W)
    packed = _sc_gather(table2, tokens3d)
    p3 = packed.reshape(_B, _L, _EMB)
    out = _tc_project(p3, token_ids, type_token_ids, W, b.reshape(1, _EMB),
                      seg_table, pos_table)
    return (out, attention_mask)


# speedup vs baseline: 2.2833x; 1.0388x over previous
import functools

import jax
import jax.numpy as jnp
from jax import lax
from jax.experimental import pallas as pl
from jax.experimental.pallas import tpu as pltpu
from jax.experimental.pallas import tpu_sc as plsc

_B = 1024
_L = 200
_FACT = 64
_EMB = 128
_TOKENS = _B * _L
_NC = 2
_NS = 16
_NW = _NC * _NS
_IDX_W = 128
_IDXROWS = _TOKENS // _IDX_W
_IDXROWS_PER_W = _IDXROWS // _NW
_ROWS_PER_W = _TOKENS // _NW
_K = 5
_GROUPS = _IDXROWS_PER_W // _K
_LANES = 16


@functools.partial(
    pl.kernel,
    out_type=jax.ShapeDtypeStruct((_TOKENS, 128), jnp.float32),
    mesh=plsc.VectorSubcoreMesh(
        core_axis_name="c", subcore_axis_name="s",
        num_cores=_NC, num_subcores=_NS),
    scratch_types=[
        pltpu.VMEM((_IDXROWS_PER_W, _IDX_W), jnp.int32),
        pltpu.VMEM((_K * _IDX_W, _EMB), jnp.float32),
        pltpu.SemaphoreType.DMA,
        pltpu.SemaphoreType.DMA,
    ],
)
def _sc_gather(table_hbm, tok_hbm, out_hbm, idx_v, sup_v, gsem, ssem):
    wid = lax.axis_index("c") * _NS + lax.axis_index("s")
    pltpu.sync_copy(tok_hbm.at[wid], idx_v)

    def to_superrow(r, carry):
        for k in range(_IDX_W // _LANES):
            sl = pl.ds(k * _LANES, _LANES)
            v = idx_v[r, sl]
            idx_v[r, sl] = lax.shift_left(
                lax.shift_right_logical(v, _VSH), _HSH) | (v & _HMASK)
        return carry

    lax.fori_loop(0, _IDXROWS_PER_W, to_superrow, 0)

    def group(grp, carry):
        j0 = grp * _K
        gh = []
        for b in range(_K):
            gh.append(pltpu.async_copy(
                table_hbm.at[idx_v.at[j0 + b]],
                sup_v.at[pl.ds(b * _IDX_W, _IDX_W)],
                gsem))
        for h in gh:
            h.wait()
        sh = []
        for b in range(_K):
            sh.append(pltpu.async_copy(
                sup_v.at[pl.ds(b * _IDX_W, _IDX_W)],
                out_hbm.at[pl.ds(wid * _ROWS_PER_W + (j0 + b) * _IDX_W,
                                 _IDX_W)],
                ssem))
        for h in sh:
            h.wait()
        return carry

    lax.fori_loop(0, _GROUPS, group, 0)


_VOCAB = 1000000
_VCHUNK = 32768
_TGRID = -(-_VOCAB // _VCHUNK)
_TROWS = _TGRID * _VCHUNK // 2
_VSH = _VCHUNK.bit_length() - 1
_HSH = _VSH - 1
_HMASK = (1 << _HSH) - 1


def _tx_body(a_ref, b_ref, o_ref):
    z = jnp.concatenate([a_ref[...], b_ref[...]], axis=0)
    o_ref[...] = z.T


_tc_transpose = pl.pallas_call(
    _tx_body,
    grid=(_TGRID,),
    in_specs=[
        pl.BlockSpec((_FACT, _VCHUNK // 2), lambda i: (0, 2 * i)),
        pl.BlockSpec((_FACT, _VCHUNK // 2),
                     lambda i: (0, jnp.minimum(
                         2 * i + 1, (_VOCAB - 1) // (_VCHUNK // 2)))),
    ],
    out_specs=pl.BlockSpec((_VCHUNK // 2, 2 * _FACT), lambda i: (i, 0)),
    out_shape=jax.ShapeDtypeStruct((_TROWS, 2 * _FACT), jnp.float32),
)


_BS = 32


def _tc_body(sup_ref, tok_ref, typ_ref, w_ref, b_ref, seg_ref, pos_ref,
             o_ref):
    sup = sup_ref[...]
    par = (lax.shift_right_logical(tok_ref[...], _HSH) & 1)[:, :, None]
    g = jnp.where(par == 1, sup[:, :, _FACT:], sup[:, :, :_FACT])
    mm = jnp.dot(g.reshape(_BS * _L, _FACT), w_ref[...],
                 preferred_element_type=jnp.float32)
    mm = mm.reshape(_BS, _L, _EMB)
    t = typ_ref[...][:, :, None]
    s0 = seg_ref[0:1, :].reshape(1, 1, _EMB)
    s1 = seg_ref[1:2, :].reshape(1, 1, _EMB)
    pos = pos_ref[...][None, :, :]
    bias = b_ref[...].reshape(1, 1, _EMB)
    o_ref[...] = mm + bias + pos + s0 + t * (s1 - s0)


_tc_project = pl.pallas_call(
    _tc_body,
    grid=(_B // _BS,),
    in_specs=[
        pl.BlockSpec((_BS, _L, _EMB), lambda i: (i, 0, 0)),
        pl.BlockSpec((_BS, _L), lambda i: (i, 0)),
        pl.BlockSpec((_BS, _L), lambda i: (i, 0)),
        pl.BlockSpec((_FACT, _EMB), lambda i: (0, 0)),
        pl.BlockSpec((1, _EMB), lambda i: (0, 0)),
        pl.BlockSpec((2, _EMB), lambda i: (0, 0)),
        pl.BlockSpec((_L, _EMB), lambda i: (0, 0)),
    ],
    out_specs=pl.BlockSpec((_BS, _L, _EMB), lambda i: (i, 0, 0)),
    out_shape=jax.ShapeDtypeStruct((_B, _L, _EMB), jnp.float32),
)


def kernel(token_ids, type_token_ids, attention_mask, tok_table, W, b,
           seg_table, pos_table):
    table_t = tok_table.T
    table2 = _tc_transpose(table_t, table_t)
    tokens3d = token_ids.reshape(_NW, _IDXROWS_PER_W, _IDX_W)
    packed = _sc_gather(table2, tokens3d)
    p3 = packed.reshape(_B, _L, _EMB)
    out = _tc_project(p3, token_ids, type_token_ids.astype(jnp.float32), W,
                      b.reshape(1, _EMB), seg_table, pos_table)
    return (out, attention_mask)


# speedup vs baseline: 2.4150x; 1.0577x over previous
import functools

import jax
import jax.numpy as jnp
from jax import lax
from jax.experimental import pallas as pl
from jax.experimental.pallas import tpu as pltpu
from jax.experimental.pallas import tpu_sc as plsc

_B = 1024
_L = 200
_FACT = 64
_EMB = 128
_TOKENS = _B * _L
_HTOK = _TOKENS // 2
_NC = 2
_NS = 16
_NW = _NC * _NS
_IDX_W = 128
_IDXROWS = _HTOK // _IDX_W
_IDXROWS_PER_W = _IDXROWS // _NW
_ROWS_PER_W = _HTOK // _NW
_K = 5
_GROUPS = _IDXROWS_PER_W // _K
_LANES = 16


@functools.partial(
    pl.kernel,
    out_type=jax.ShapeDtypeStruct((_HTOK, 128), jnp.float32),
    mesh=plsc.VectorSubcoreMesh(
        core_axis_name="c", subcore_axis_name="s",
        num_cores=_NC, num_subcores=_NS),
    scratch_types=[
        pltpu.VMEM((_IDXROWS_PER_W, _IDX_W), jnp.int32),
        pltpu.VMEM((_K * _IDX_W, _EMB), jnp.float32),
        pltpu.SemaphoreType.DMA,
        pltpu.SemaphoreType.DMA,
    ],
)
def _sc_gather(table_hbm, tok_hbm, out_hbm, idx_v, sup_v, gsem, ssem):
    wid = lax.axis_index("c") * _NS + lax.axis_index("s")
    pltpu.sync_copy(tok_hbm.at[wid], idx_v)

    def to_superrow(r, carry):
        for k in range(_IDX_W // _LANES):
            sl = pl.ds(k * _LANES, _LANES)
            v = idx_v[r, sl]
            idx_v[r, sl] = lax.shift_left(
                lax.shift_right_logical(v, _VSH), _HSH) | (v & _HMASK)
        return carry

    lax.fori_loop(0, _IDXROWS_PER_W, to_superrow, 0)

    def group(grp, carry):
        j0 = grp * _K
        gh = []
        for b in range(_K):
            gh.append(pltpu.async_copy(
                table_hbm.at[idx_v.at[j0 + b]],
                sup_v.at[pl.ds(b * _IDX_W, _IDX_W)],
                gsem))
        for h in gh:
            h.wait()
        sh = []
        for b in range(_K):
            sh.append(pltpu.async_copy(
                sup_v.at[pl.ds(b * _IDX_W, _IDX_W)],
                out_hbm.at[pl.ds(wid * _ROWS_PER_W + (j0 + b) * _IDX_W,
                                 _IDX_W)],
                ssem))
        for h in sh:
            h.wait()
        return carry

    lax.fori_loop(0, _GROUPS, group, 0)


_VOCAB = 1000000
_VCHUNK = 32768
_TGRID = -(-_VOCAB // _VCHUNK)
_TROWS = _TGRID * _VCHUNK // 2
_VSH = _VCHUNK.bit_length() - 1
_HSH = _VSH - 1
_HMASK = (1 << _HSH) - 1


def _tx_body(a_ref, b_ref, o_ref):
    z = jnp.concatenate([a_ref[...], b_ref[...]], axis=0)
    o_ref[...] = z.T


_tc_transpose = pl.pallas_call(
    _tx_body,
    grid=(_TGRID,),
    in_specs=[
        pl.BlockSpec((_FACT, _VCHUNK // 2), lambda i: (0, 2 * i)),
        pl.BlockSpec((_FACT, _VCHUNK // 2),
                     lambda i: (0, jnp.minimum(
                         2 * i + 1, (_VOCAB - 1) // (_VCHUNK // 2)))),
    ],
    out_specs=pl.BlockSpec((_VCHUNK // 2, 2 * _FACT), lambda i: (i, 0)),
    out_shape=jax.ShapeDtypeStruct((_TROWS, 2 * _FACT), jnp.float32),
)


_BS = 32


def _tc_body(sup_ref, tok_ref, typ_ref, w_ref, b_ref, seg_ref, pos_ref,
             o_ref):
    sup = sup_ref[...]
    par = (lax.shift_right_logical(tok_ref[...], _HSH) & 1)[:, :, None]
    g = jnp.where(par == 1, sup[:, :, _FACT:], sup[:, :, :_FACT])
    mm = jnp.dot(g.reshape(_BS * _L, _FACT), w_ref[...],
                 preferred_element_type=jnp.float32)
    mm = mm.reshape(_BS, _L, _EMB)
    t = typ_ref[...][:, :, None]
    s0 = seg_ref[0:1, :].reshape(1, 1, _EMB)
    s1 = seg_ref[1:2, :].reshape(1, 1, _EMB)
    pos = pos_ref[...][None, :, :]
    bias = b_ref[...].reshape(1, 1, _EMB)
    o_ref[...] = mm + bias + pos + s0 + t * (s1 - s0)


def _tc_body_alias(sup_ref, tok_ref, typ_ref, w_ref, b_ref, seg_ref,
                   pos_ref, prev_ref, o_ref):
    del prev_ref
    _tc_body(sup_ref, tok_ref, typ_ref, w_ref, b_ref, seg_ref, pos_ref,
             o_ref)


_HB = _B // 2
_HGRID = _HB // _BS


def _mk_project(off, alias):
    specs = [
        pl.BlockSpec((_BS, _L, _EMB), lambda i: (i, 0, 0)),
        pl.BlockSpec((_BS, _L), lambda i: (i, 0)),
        pl.BlockSpec((_BS, _L), lambda i: (i, 0)),
        pl.BlockSpec((_FACT, _EMB), lambda i: (0, 0)),
        pl.BlockSpec((1, _EMB), lambda i: (0, 0)),
        pl.BlockSpec((2, _EMB), lambda i: (0, 0)),
        pl.BlockSpec((_L, _EMB), lambda i: (0, 0)),
    ]
    body = _tc_body
    kwargs = {}
    if alias:
        specs = specs + [pl.BlockSpec(memory_space=pl.ANY)]
        body = _tc_body_alias
        kwargs["input_output_aliases"] = {7: 0}
    return pl.pallas_call(
        body,
        grid=(_HGRID,),
        in_specs=specs,
        out_specs=pl.BlockSpec((_BS, _L, _EMB), lambda i: (i + off, 0, 0)),
        out_shape=jax.ShapeDtypeStruct((_B, _L, _EMB), jnp.float32),
        **kwargs)


_tc_project_lo = _mk_project(0, False)
_tc_project_hi = _mk_project(_HGRID, True)


def kernel(token_ids, type_token_ids, attention_mask, tok_table, W, b,
           seg_table, pos_table):
    table_t = tok_table.T
    table2 = _tc_transpose(table_t, table_t)
    tok_lo, tok_hi = token_ids[:_HB], token_ids[_HB:]
    g_lo = _sc_gather(table2, tok_lo.reshape(_NW, _IDXROWS_PER_W, _IDX_W))
    g_hi = _sc_gather(table2, tok_hi.reshape(_NW, _IDXROWS_PER_W, _IDX_W))
    typf = type_token_ids.astype(jnp.float32)
    b2 = b.reshape(1, _EMB)
    out1 = _tc_project_lo(g_lo.reshape(_HB, _L, _EMB), tok_lo, typf[:_HB],
                          W, b2, seg_table, pos_table)
    out = _tc_project_hi(g_hi.reshape(_HB, _L, _EMB), tok_hi, typf[_HB:],
                         W, b2, seg_table, pos_table, out1)
    return (out, attention_mask)


# speedup vs baseline: 2.4152x; 1.0001x over previous
import functools

import jax
import jax.numpy as jnp
from jax import lax
from jax.experimental import pallas as pl
from jax.experimental.pallas import tpu as pltpu
from jax.experimental.pallas import tpu_sc as plsc

_B = 1024
_L = 200
_FACT = 64
_EMB = 128
_TOKENS = _B * _L
_HTOK = _TOKENS // 2
_NC = 2
_NS = 16
_NW = _NC * _NS
_IDX_W = 128
_IDXROWS = _HTOK // _IDX_W
_IDXROWS_PER_W = _IDXROWS // _NW
_ROWS_PER_W = _HTOK // _NW
_K = 5
_GROUPS = _IDXROWS_PER_W // _K
_LANES = 16


@functools.partial(
    pl.kernel,
    out_type=jax.ShapeDtypeStruct((_HTOK, 128), jnp.float32),
    mesh=plsc.VectorSubcoreMesh(
        core_axis_name="c", subcore_axis_name="s",
        num_cores=_NC, num_subcores=_NS),
    scratch_types=[
        pltpu.VMEM((_IDXROWS_PER_W, _IDX_W), jnp.int32),
        pltpu.VMEM((_K * _IDX_W, _EMB), jnp.float32),
        pltpu.SemaphoreType.DMA,
        pltpu.SemaphoreType.DMA,
    ],
)
def _sc_gather(table_hbm, tok_hbm, out_hbm, idx_v, sup_v, gsem, ssem):
    wid = lax.axis_index("c") * _NS + lax.axis_index("s")
    pltpu.sync_copy(tok_hbm.at[wid], idx_v)

    def to_superrow(r, carry):
        for k in range(_IDX_W // _LANES):
            sl = pl.ds(k * _LANES, _LANES)
            v = idx_v[r, sl]
            idx_v[r, sl] = lax.shift_left(
                lax.shift_right_logical(v, _VSH), _HSH) | (v & _HMASK)
        return carry

    lax.fori_loop(0, _IDXROWS_PER_W, to_superrow, 0)

    def group(grp, carry):
        j0 = grp * _K
        gh = []
        for b in range(_K):
            gh.append(pltpu.async_copy(
                table_hbm.at[idx_v.at[j0 + b]],
                sup_v.at[pl.ds(b * _IDX_W, _IDX_W)],
                gsem))
        for h in gh:
            h.wait()
        sh = []
        for b in range(_K):
            sh.append(pltpu.async_copy(
                sup_v.at[pl.ds(b * _IDX_W, _IDX_W)],
                out_hbm.at[pl.ds(wid * _ROWS_PER_W + (j0 + b) * _IDX_W,
                                 _IDX_W)],
                ssem))
        for h in sh:
            h.wait()
        return carry

    lax.fori_loop(0, _GROUPS, group, 0)


_VOCAB = 1000000
_VCHUNK = 32768
_TGRID = -(-_VOCAB // _VCHUNK)
_TROWS = _TGRID * _VCHUNK // 2
_VSH = _VCHUNK.bit_length() - 1
_HSH = _VSH - 1
_HMASK = (1 << _HSH) - 1


def _tx_body(a_ref, b_ref, o_ref):
    z = jnp.concatenate([a_ref[...], b_ref[...]], axis=0)
    o_ref[...] = z.T


_tc_transpose = pl.pallas_call(
    _tx_body,
    grid=(_TGRID,),
    in_specs=[
        pl.BlockSpec((_FACT, _VCHUNK // 2), lambda i: (0, 2 * i)),
        pl.BlockSpec((_FACT, _VCHUNK // 2),
                     lambda i: (0, jnp.minimum(
                         2 * i + 1, (_VOCAB - 1) // (_VCHUNK // 2)))),
    ],
    out_specs=pl.BlockSpec((_VCHUNK // 2, 2 * _FACT), lambda i: (i, 0)),
    out_shape=jax.ShapeDtypeStruct((_TROWS, 2 * _FACT), jnp.float32),
)


_BS = 64


def _tc_body(sup_ref, tok_ref, typ_ref, w_ref, b_ref, seg_ref, pos_ref,
             o_ref):
    sup = sup_ref[...]
    par = (lax.shift_right_logical(tok_ref[...], _HSH) & 1)[:, :, None]
    g = jnp.where(par == 1, sup[:, :, _FACT:], sup[:, :, :_FACT])
    mm = jnp.dot(g.reshape(_BS * _L, _FACT), w_ref[...],
                 preferred_element_type=jnp.float32)
    mm = mm.reshape(_BS, _L, _EMB)
    t = typ_ref[...][:, :, None]
    s0 = seg_ref[0:1, :].reshape(1, 1, _EMB)
    s1 = seg_ref[1:2, :].reshape(1, 1, _EMB)
    pos = pos_ref[...][None, :, :]
    bias = b_ref[...].reshape(1, 1, _EMB)
    o_ref[...] = mm + bias + pos + s0 + t * (s1 - s0)


def _tc_body_alias(sup_ref, tok_ref, typ_ref, w_ref, b_ref, seg_ref,
                   pos_ref, prev_ref, o_ref):
    del prev_ref
    _tc_body(sup_ref, tok_ref, typ_ref, w_ref, b_ref, seg_ref, pos_ref,
             o_ref)


_HB = _B // 2
_HGRID = _HB // _BS


def _mk_project(off, alias):
    specs = [
        pl.BlockSpec((_BS, _L, _EMB), lambda i: (i, 0, 0)),
        pl.BlockSpec((_BS, _L), lambda i: (i, 0)),
        pl.BlockSpec((_BS, _L), lambda i: (i, 0)),
        pl.BlockSpec((_FACT, _EMB), lambda i: (0, 0)),
        pl.BlockSpec((1, _EMB), lambda i: (0, 0)),
        pl.BlockSpec((2, _EMB), lambda i: (0, 0)),
        pl.BlockSpec((_L, _EMB), lambda i: (0, 0)),
    ]
    body = _tc_body
    kwargs = {}
    if alias:
        specs = specs + [pl.BlockSpec(memory_space=pl.ANY)]
        body = _tc_body_alias
        kwargs["input_output_aliases"] = {7: 0}
    return pl.pallas_call(
        body,
        grid=(_HGRID,),
        in_specs=specs,
        out_specs=pl.BlockSpec((_BS, _L, _EMB), lambda i: (i + off, 0, 0)),
        out_shape=jax.ShapeDtypeStruct((_B, _L, _EMB), jnp.float32),
        **kwargs)


_tc_project_lo = _mk_project(0, False)
_tc_project_hi = _mk_project(_HGRID, True)


def kernel(token_ids, type_token_ids, attention_mask, tok_table, W, b,
           seg_table, pos_table):
    table_t = tok_table.T
    table2 = _tc_transpose(table_t, table_t)
    tok_lo, tok_hi = token_ids[:_HB], token_ids[_HB:]
    g_lo = _sc_gather(table2, tok_lo.reshape(_NW, _IDXROWS_PER_W, _IDX_W))
    g_hi = _sc_gather(table2, tok_hi.reshape(_NW, _IDXROWS_PER_W, _IDX_W))
    typf = type_token_ids.astype(jnp.float32)
    b2 = b.reshape(1, _EMB)
    out1 = _tc_project_lo(g_lo.reshape(_HB, _L, _EMB), tok_lo, typf[:_HB],
                          W, b2, seg_table, pos_table)
    out = _tc_project_hi(g_hi.reshape(_HB, _L, _EMB), tok_hi, typf[_HB:],
                         W, b2, seg_table, pos_table, out1)
    return (out, attention_mask)
